# Initial kernel scaffold; baseline (speedup 1.0000x reference)
import jax
import jax.numpy as jnp
from jax.experimental import pallas as pl


def kernel(x, tables):
    raise NotImplementedError("write your pallas kernel here")


# speedup vs baseline: 21.9040x; 21.9040x over previous
import functools
import jax
import jax.numpy as jnp
from jax import lax
from jax.experimental import pallas as pl
from jax.experimental.pallas import tpu as pltpu
from jax.experimental.pallas import tpu_sc as plsc

_F = 26
_D = 32
_ROWS = 4000
_TOTAL = _F * _ROWS
_NPAIR = (_F * (_F - 1)) // 2


def kernel(x, tables):
    B = x.shape[0]
    offs = (jnp.arange(_F, dtype=x.dtype) * _ROWS)[None, :]
    xiT = (x + offs).T
    flat_tables = tables.reshape(_F * _TOTAL, _D)

    info = plsc.get_sparse_core_info()
    NC, NS = info.num_cores, info.num_subcores
    NW = NC * NS
    Bw = B // NW

    mesh = plsc.VectorSubcoreMesh(core_axis_name="c", subcore_axis_name="s")

    @functools.partial(
        pl.kernel,
        mesh=mesh,
        compiler_params=pltpu.CompilerParams(use_tc_tiling_on_sc=False),
        out_type=jax.ShapeDtypeStruct((B, _NPAIR * _D), jnp.float32),
        scratch_types=[
            pltpu.VMEM((_F, Bw), jnp.int32),
            pltpu.VMEM((Bw,), jnp.int32),
            pltpu.VMEM((Bw,), jnp.int32),
            pltpu.VMEM((Bw, _D), jnp.float32),
            pltpu.VMEM((Bw, _D), jnp.float32),
            pltpu.VMEM((Bw, _D), jnp.float32),
            pltpu.SemaphoreType.DMA,
            pltpu.SemaphoreType.DMA,
        ],
    )
    def k(xiT_hbm, tab_hbm, out_hbm, xibuf, idxa, idxb, abuf, bbuf, cbuf,
          sema, semb):
        wid = lax.axis_index("s") * NC + lax.axis_index("c")
        base = wid * Bw
        pltpu.sync_copy(xiT_hbm.at[:, pl.ds(base, Bw)], xibuf)

        def pair_body(p, carry):
            r = p // 25
            c = p % 25
            upper = c >= r
            i = jnp.where(upper, r, 25 - r)
            j = jnp.where(upper, c + 1, 25 - c)

            ioff = i * _TOTAL
            joff = j * _TOTAL

            def mk_idx(kk, _):
                idxa[pl.ds(kk * 16, 16)] = xibuf[j, pl.ds(kk * 16, 16)] + ioff
                idxb[pl.ds(kk * 16, 16)] = xibuf[i, pl.ds(kk * 16, 16)] + joff
                return 0

            lax.fori_loop(0, Bw // 16, mk_idx, 0)

            cpa = pltpu.async_copy(tab_hbm.at[idxa], abuf, sema)
            cpb = pltpu.async_copy(tab_hbm.at[idxb], bbuf, semb)
            cpa.wait()
            cpb.wait()

            def mul_row(rr, _):
                cbuf[rr, pl.ds(0, 16)] = (
                    abuf[rr, pl.ds(0, 16)] * bbuf[rr, pl.ds(0, 16)])
                cbuf[rr, pl.ds(16, 16)] = (
                    abuf[rr, pl.ds(16, 16)] * bbuf[rr, pl.ds(16, 16)])
                return 0

            lax.fori_loop(0, Bw, mul_row, 0)

            pref = (i * (2 * _F - 1 - i)) // 2 + j - i - 1
            pltpu.sync_copy(cbuf, out_hbm.at[pl.ds(base, Bw),
                                             pl.ds(pref * _D, _D)])
            return carry

        lax.fori_loop(0, _NPAIR, pair_body, 0)

    out = k(xiT, flat_tables)
    return out.reshape(B, _NPAIR, _D)


# speedup vs baseline: 22.8678x; 1.0440x over previous
import functools
import jax
import jax.numpy as jnp
from jax import lax
from jax.experimental import pallas as pl
from jax.experimental.pallas import tpu as pltpu
from jax.experimental.pallas import tpu_sc as plsc

_F = 26
_D = 32
_ROWS = 4000
_TOTAL = _F * _ROWS
_NPAIR = (_F * (_F - 1)) // 2
_NBUF = 5
_NGROUP = _NPAIR // _NBUF


def _pair_from_flat(p):
    r = p // 25
    c = p % 25
    upper = c >= r
    i = jnp.where(upper, r, 25 - r)
    j = jnp.where(upper, c + 1, 25 - c)
    return i, j


def kernel(x, tables):
    B = x.shape[0]
    offs = (jnp.arange(_F, dtype=x.dtype) * _ROWS)[None, :]
    xiT = (x + offs).T
    flat_tables = tables.reshape(_F * _TOTAL, _D)

    info = plsc.get_sparse_core_info()
    NC, NS = info.num_cores, info.num_subcores
    NW = NC * NS
    Bw = B // NW

    mesh = plsc.VectorSubcoreMesh(core_axis_name="c", subcore_axis_name="s")

    scratch = [pltpu.VMEM((_F, Bw), jnp.int32)]
    for _ in range(_NBUF):
        scratch += [
            pltpu.VMEM((Bw,), jnp.int32),
            pltpu.VMEM((Bw,), jnp.int32),
            pltpu.VMEM((Bw, _D), jnp.float32),
            pltpu.VMEM((Bw, _D), jnp.float32),
            pltpu.VMEM((Bw, _D), jnp.float32),
        ]
    scratch += [pltpu.SemaphoreType.DMA] * (3 * _NBUF)

    @functools.partial(
        pl.kernel,
        mesh=mesh,
        compiler_params=pltpu.CompilerParams(use_tc_tiling_on_sc=False),
        out_type=jax.ShapeDtypeStruct((B, _NPAIR * _D), jnp.float32),
        scratch_types=scratch,
    )
    def k(xiT_hbm, tab_hbm, out_hbm, xibuf, *rest):
        idxa = [rest[5 * u + 0] for u in range(_NBUF)]
        idxb = [rest[5 * u + 1] for u in range(_NBUF)]
        abuf = [rest[5 * u + 2] for u in range(_NBUF)]
        bbuf = [rest[5 * u + 3] for u in range(_NBUF)]
        cbuf = [rest[5 * u + 4] for u in range(_NBUF)]
        sema = rest[5 * _NBUF + 0 * _NBUF:5 * _NBUF + 1 * _NBUF]
        semb = rest[5 * _NBUF + 1 * _NBUF:5 * _NBUF + 2 * _NBUF]
        semw = rest[5 * _NBUF + 2 * _NBUF:5 * _NBUF + 3 * _NBUF]

        wid = lax.axis_index("s") * NC + lax.axis_index("c")
        base = wid * Bw
        pltpu.sync_copy(xiT_hbm.at[:, pl.ds(base, Bw)], xibuf)

        def fire(u, p):
            i, j = _pair_from_flat(p)
            ioff = i * _TOTAL
            joff = j * _TOTAL
            for kk in range(Bw // 16):
                sl = pl.ds(kk * 16, 16)
                idxa[u][sl] = xibuf[j, sl] + ioff
                idxb[u][sl] = xibuf[i, sl] + joff
            pltpu.async_copy(tab_hbm.at[idxa[u]], abuf[u], sema[u])
            pltpu.async_copy(tab_hbm.at[idxb[u]], bbuf[u], semb[u])

        def out_slot(p):
            i, j = _pair_from_flat(p)
            pref = (i * (2 * _F - 1 - i)) // 2 + j - i - 1
            return out_hbm.at[pl.ds(base, Bw), pl.ds(pref * _D, _D)]

        for u in range(_NBUF):
            fire(u, u)

        def group_body(g, carry):
            p0 = g * _NBUF
            for u in range(_NBUF):
                p = p0 + u
                pltpu.make_async_copy(
                    tab_hbm.at[idxa[u]], abuf[u], sema[u]).wait()
                pltpu.make_async_copy(
                    tab_hbm.at[idxb[u]], bbuf[u], semb[u]).wait()

                @pl.when(g > 0)
                def _():
                    pltpu.make_async_copy(cbuf[u], out_slot(p), semw[u]).wait()

                def mul_row(rr, _):
                    cbuf[u][rr, pl.ds(0, 16)] = (
                        abuf[u][rr, pl.ds(0, 16)] * bbuf[u][rr, pl.ds(0, 16)])
                    cbuf[u][rr, pl.ds(16, 16)] = (
                        abuf[u][rr, pl.ds(16, 16)] * bbuf[u][rr, pl.ds(16, 16)])
                    return 0

                lax.fori_loop(0, Bw, mul_row, 0, unroll=8)

                pltpu.async_copy(cbuf[u], out_slot(p), semw[u])

                @pl.when(g < _NGROUP - 1)
                def _():
                    fire(u, p + _NBUF)
            return carry

        lax.fori_loop(0, _NGROUP, group_body, 0)

        for u in range(_NBUF):
            pltpu.make_async_copy(cbuf[u], out_slot(0), semw[u]).wait()

    out = k(xiT, flat_tables)
    return out.reshape(B, _NPAIR, _D)
